# baseline (speedup 1.0000x reference)
import jax
import jax.numpy as jnp
from jax import lax
from jax.experimental import pallas as pl
from jax.experimental.pallas import tpu as pltpu
from jax.experimental.pallas import tpu_sc as plsc

N_NODES = 10000
N_EDGES = 320000
NPAD = 10240
DUMMY = N_NODES
NC, NS = 2, 16
NW = NC * NS
CHUNK = 128
C_REAL = 80
NBUF = 4
RS = 8
LOOK = 3
C_ARR = C_REAL + RS
RPT = NPAD // NS
BLK = 400

_mesh = plsc.VectorSubcoreMesh(core_axis_name="c", subcore_axis_name="s")


def _deg_body(e_hbm, ones_hbm, zeros_hbm, out_hbm,
              e_v, ones_v, acc_o, acc_i, sem_s):
    c = lax.axis_index("c")
    s = lax.axis_index("s")
    wid = s * NC + c
    pltpu.sync_copy(e_hbm.at[wid, pl.ds(0, C_REAL)], e_v)
    pltpu.sync_copy(ones_hbm, ones_v)
    sl = pl.ds(s * RPT, RPT)
    pltpu.sync_copy(zeros_hbm.at[sl], acc_o.at[sl])
    pltpu.sync_copy(zeros_hbm.at[sl], acc_i.at[sl])
    plsc.subcore_barrier()

    def fire(j, carry):
        pltpu.async_copy(ones_v, acc_o.at[e_v.at[j, 0]], sem_s, add=True)
        pltpu.async_copy(ones_v, acc_i.at[e_v.at[j, 1]], sem_s, add=True)
        return carry

    lax.fori_loop(0, C_REAL, fire, 0)

    def drain(j, carry):
        pltpu.make_async_copy(ones_v, acc_o.at[e_v.at[j, 0]], sem_s).wait()
        pltpu.make_async_copy(ones_v, acc_i.at[e_v.at[j, 1]], sem_s).wait()
        return carry

    lax.fori_loop(0, C_REAL, drain, 0)
    plsc.subcore_barrier()
    pltpu.sync_copy(acc_o.at[sl], out_hbm.at[c, 0, sl])
    pltpu.sync_copy(acc_i.at[sl], out_hbm.at[c, 1, sl])


_deg_call = pl.kernel(
    _deg_body,
    out_type=jax.ShapeDtypeStruct((NC, 2, NPAD, 8), jnp.float32),
    mesh=_mesh,
    scratch_types=[
        pltpu.VMEM((C_REAL, 2, CHUNK), jnp.int32),
        pltpu.VMEM((CHUNK, 8), jnp.float32),
        pltpu.VMEM_SHARED((NPAD, 8), jnp.float32),
        pltpu.VMEM_SHARED((NPAD, 8), jnp.float32),
        pltpu.SemaphoreType.DMA,
    ],
)


def _make_agg(d, npass):
    def _agg_body(h_hbm, e_hbm, zeros_hbm, out_hbm,
                  ring, rows, h_sp, acc,
                  se0, se1, se2, se3, se4, se5, se6, se7,
                  sg0, sg1, sg2, sg3, ss0, ss1, ss2, ss3):
        c = lax.axis_index("c")
        s = lax.axis_index("s")
        wid = s * NC + c
        sems_e = (se0, se1, se2, se3, se4, se5, se6, se7)
        sems_g = (sg0, sg1, sg2, sg3)
        sems_s = (ss0, ss1, ss2, ss3)
        sl = pl.ds(s * RPT, RPT)

        def phase(j0, k, first):
            b = k % NBUF
            bn = (k + LOOK) % NBUF
            rn = (k + LOOK) % RS
            pltpu.make_async_copy(h_sp.at[ring.at[k, 0]], rows.at[b],
                                  sems_g[b]).wait()
            pltpu.async_copy(rows.at[b], acc.at[ring.at[k, 1]],
                             sems_s[b], add=True)
            if not first:
                pltpu.make_async_copy(rows.at[bn],
                                      acc.at[ring.at[(k - 1) % RS, 1]],
                                      sems_s[bn]).wait()
            pltpu.async_copy(e_hbm.at[wid, j0 + k + RS - 1],
                             ring.at[(k - 1) % RS],
                             sems_e[(k - 1) % RS])
            pltpu.make_async_copy(e_hbm.at[wid, j0 + k + LOOK], ring.at[rn],
                                  sems_e[rn]).wait()
            pltpu.async_copy(h_sp.at[ring.at[rn, 0]], rows.at[bn], sems_g[bn])

        for p in range(npass):
            pltpu.sync_copy(h_hbm.at[p, sl], h_sp.at[sl])
            pltpu.sync_copy(zeros_hbm.at[sl], acc.at[sl])
            for k in range(RS - 1):
                pltpu.async_copy(e_hbm.at[wid, k], ring.at[k], sems_e[k])
            plsc.subcore_barrier()
            for b in range(LOOK):
                pltpu.make_async_copy(e_hbm.at[wid, b], ring.at[b],
                                      sems_e[b]).wait()
                pltpu.async_copy(h_sp.at[ring.at[b, 0]], rows.at[b], sems_g[b])

            for k in range(RS):
                phase(0, k, first=(k == 0))

            def body(u, carry):
                j0 = u * RS
                for k in range(RS):
                    phase(j0, k, first=False)
                return carry

            lax.fori_loop(1, C_REAL // RS, body, 0)

            for i in range(LOOK):
                b = (C_REAL + i) % NBUF
                pltpu.make_async_copy(h_sp.at[ring.at[b, 0]], rows.at[b],
                                      sems_g[b]).wait()
            bl = (C_REAL - 1) % NBUF
            pltpu.make_async_copy(rows.at[bl], acc.at[ring.at[0, 1]],
                                  sems_s[bl]).wait()
            for i in range(LOOK, RS - 1):
                k = (C_REAL + i) % RS
                pltpu.make_async_copy(e_hbm.at[wid, k], ring.at[k],
                                      sems_e[k]).wait()
            plsc.subcore_barrier()
            pltpu.sync_copy(acc.at[sl], out_hbm.at[c, p, sl])

    params = None
    if d % 128 != 0:
        params = pltpu.CompilerParams(use_tc_tiling_on_sc=False)
    return pl.kernel(
        _agg_body,
        out_type=jax.ShapeDtypeStruct((NC, npass, NPAD, d), jnp.float32),
        mesh=_mesh,
        compiler_params=params,
        scratch_types=[
            pltpu.VMEM((RS, 2, CHUNK), jnp.int32),
            pltpu.VMEM((NBUF, CHUNK, d), jnp.float32),
            pltpu.VMEM_SHARED((NPAD, d), jnp.float32),
            pltpu.VMEM_SHARED((NPAD, d), jnp.float32),
        ] + [pltpu.SemaphoreType.DMA] * (RS + 2 * NBUF),
    )


_agg64x2 = _make_agg(64, 2)
_agg40 = _make_agg(40, 1)


def _tc1_body(x_ref, w_ref, deg_ref, o_ref):
    d = jnp.sum(deg_ref[...], axis=(0, 1))
    dsum = jnp.sum(d, axis=1, keepdims=True)
    cs = lax.rsqrt(jnp.maximum(dsum, 1.0))
    h = jnp.dot(x_ref[...], w_ref[...], preferred_element_type=jnp.float32)
    h = h * cs
    o_ref[...] = jnp.stack([h[:, :64], h[:, 64:]], axis=0)


def _tc1(feats_p, w1, degs):
    return pl.pallas_call(
        _tc1_body,
        grid=(N_NODES // BLK,),
        in_specs=[
            pl.BlockSpec((BLK, 128), lambda i: (i, 0)),
            pl.BlockSpec((128, 128), lambda i: (0, 0)),
            pl.BlockSpec((NC, 1, BLK, 8), lambda i: (0, 0, i, 0)),
        ],
        out_specs=pl.BlockSpec((2, BLK, 64), lambda i: (0, i, 0)),
        out_shape=jax.ShapeDtypeStruct((2, NPAD, 64), jnp.float32),
    )(feats_p, w1, degs)


def _tc2_body(a_ref, deg_ref, b1_ref, w2_ref, o_ref):
    d = deg_ref[...]
    do = jnp.sum(jnp.sum(d[:, 0], axis=0), axis=1, keepdims=True)
    di = jnp.sum(jnp.sum(d[:, 1], axis=0), axis=1, keepdims=True)
    cs = lax.rsqrt(jnp.maximum(do, 1.0))
    ci = lax.rsqrt(jnp.maximum(di, 1.0))
    a = a_ref[...]
    agg = jnp.concatenate([a[0, 0] + a[1, 0], a[0, 1] + a[1, 1]], axis=-1)
    t = jnp.maximum(agg * ci + b1_ref[...], 0.0)
    h2 = jnp.dot(t, w2_ref[...], preferred_element_type=jnp.float32)
    o_ref[...] = h2 * cs


def _tc2(a1, degs, b1r, w2):
    return pl.pallas_call(
        _tc2_body,
        grid=(N_NODES // BLK,),
        in_specs=[
            pl.BlockSpec((NC, 2, BLK, 64), lambda i: (0, 0, i, 0)),
            pl.BlockSpec((NC, 2, BLK, 8), lambda i: (0, 0, i, 0)),
            pl.BlockSpec((1, 128), lambda i: (0, 0)),
            pl.BlockSpec((128, 40), lambda i: (0, 0)),
        ],
        out_specs=pl.BlockSpec((BLK, 40), lambda i: (i, 0)),
        out_shape=jax.ShapeDtypeStruct((NPAD, 40), jnp.float32),
    )(a1, degs, b1r, w2)


def _tc3_body(a_ref, deg_ref, b2_ref, o_ref):
    d = jnp.sum(deg_ref[...], axis=(0, 1))
    dsum = jnp.sum(d, axis=1, keepdims=True)
    ci = lax.rsqrt(jnp.maximum(dsum, 1.0))
    a = a_ref[...]
    o_ref[...] = (a[0, 0] + a[1, 0]) * ci + b2_ref[...]


def _tc3(a2, degs, b2r):
    return pl.pallas_call(
        _tc3_body,
        grid=(N_NODES // BLK,),
        in_specs=[
            pl.BlockSpec((NC, 1, BLK, 40), lambda i: (0, 0, i, 0)),
            pl.BlockSpec((NC, 1, BLK, 8), lambda i: (0, 1, i, 0)),
            pl.BlockSpec((1, 40), lambda i: (0, 0)),
        ],
        out_specs=pl.BlockSpec((BLK, 40), lambda i: (i, 0)),
        out_shape=jax.ShapeDtypeStruct((N_NODES, 40), jnp.float32),
    )(a2, degs, b2r)


def _pack_edges(e32):
    pad_n = C_REAL * NW * CHUNK - N_EDGES
    pad = jnp.full((2, pad_n), DUMMY, jnp.int32)
    main = jnp.concatenate([e32, pad], axis=1).reshape(2, NW, C_REAL, CHUNK)
    main = jnp.transpose(main, (1, 2, 0, 3))
    filler = jnp.full((NW, C_ARR - C_REAL, 2, CHUNK), DUMMY, jnp.int32)
    return jnp.concatenate([main, filler], axis=1)


def kernel(feats, edge_index, W1, b1, W2, b2):
    e3 = _pack_edges(edge_index.astype(jnp.int32))

    b1r = b1.reshape(1, 128)
    b2r = b2.reshape(1, 40)

    ones8 = jnp.tile(jnp.array([1, 0, 0, 0, 0, 0, 0, 0], jnp.float32), (CHUNK, 1))
    zeros8 = jnp.zeros((NPAD, 8), jnp.float32)
    zeros64 = jnp.zeros((NPAD, 64), jnp.float32)
    zeros40 = jnp.zeros((NPAD, 40), jnp.float32)

    degs = _deg_call(e3, ones8, zeros8)
    h1 = _tc1(feats, W1, degs)
    a1 = _agg64x2(h1, e3, zeros64)
    h2 = _tc2(a1, degs, b1r, W2)
    a2 = _agg40(h2.reshape(1, NPAD, 40), e3, zeros40)
    return _tc3(a2, degs, b2r)


# speedup vs baseline: 1.0154x; 1.0154x over previous
import jax
import jax.numpy as jnp
from jax import lax
from jax.experimental import pallas as pl
from jax.experimental.pallas import tpu as pltpu
from jax.experimental.pallas import tpu_sc as plsc

N_NODES = 10000
N_EDGES = 320000
NPAD = 10240
DUMMY = N_NODES
NC, NS = 2, 16
NW = NC * NS
CHUNK = 128
C_REAL = 80
NBUF = 4
RS = 8
LOOK = 3
C_ARR = C_REAL + RS
RPT = NPAD // NS
BLK = 512

_mesh = plsc.VectorSubcoreMesh(core_axis_name="c", subcore_axis_name="s")


def _deg_body(e_hbm, ones_hbm, zeros_hbm, out_hbm,
              e_v, ones_v, acc_o, acc_i, sem_s):
    c = lax.axis_index("c")
    s = lax.axis_index("s")
    wid = s * NC + c
    pltpu.sync_copy(e_hbm.at[wid, pl.ds(0, C_REAL)], e_v)
    pltpu.sync_copy(ones_hbm, ones_v)
    sl = pl.ds(s * RPT, RPT)
    pltpu.sync_copy(zeros_hbm.at[sl], acc_o.at[sl])
    pltpu.sync_copy(zeros_hbm.at[sl], acc_i.at[sl])
    plsc.subcore_barrier()

    def fire(j, carry):
        pltpu.async_copy(ones_v, acc_o.at[e_v.at[j, 0]], sem_s, add=True)
        pltpu.async_copy(ones_v, acc_i.at[e_v.at[j, 1]], sem_s, add=True)
        return carry

    lax.fori_loop(0, C_REAL, fire, 0)

    def drain(j, carry):
        pltpu.make_async_copy(ones_v, acc_o.at[e_v.at[j, 0]], sem_s).wait()
        pltpu.make_async_copy(ones_v, acc_i.at[e_v.at[j, 1]], sem_s).wait()
        return carry

    lax.fori_loop(0, C_REAL, drain, 0)
    plsc.subcore_barrier()
    pltpu.sync_copy(acc_o.at[sl], out_hbm.at[c, 0, sl])
    pltpu.sync_copy(acc_i.at[sl], out_hbm.at[c, 1, sl])


_deg_call = pl.kernel(
    _deg_body,
    out_type=jax.ShapeDtypeStruct((NC, 2, NPAD, 8), jnp.float32),
    mesh=_mesh,
    scratch_types=[
        pltpu.VMEM((C_REAL, 2, CHUNK), jnp.int32),
        pltpu.VMEM((CHUNK, 8), jnp.float32),
        pltpu.VMEM_SHARED((NPAD, 8), jnp.float32),
        pltpu.VMEM_SHARED((NPAD, 8), jnp.float32),
        pltpu.SemaphoreType.DMA,
    ],
)


def _make_agg(d, npass):
    def _agg_body(h_hbm, e_hbm, zeros_hbm, out_hbm,
                  ring, rows, h_sp, acc,
                  se0, se1, se2, se3, se4, se5, se6, se7,
                  sg0, sg1, sg2, sg3, ss0, ss1, ss2, ss3):
        c = lax.axis_index("c")
        s = lax.axis_index("s")
        wid = s * NC + c
        sems_e = (se0, se1, se2, se3, se4, se5, se6, se7)
        sems_g = (sg0, sg1, sg2, sg3)
        sems_s = (ss0, ss1, ss2, ss3)
        sl = pl.ds(s * RPT, RPT)

        def phase(j0, k, first):
            b = k % NBUF
            bn = (k + LOOK) % NBUF
            rn = (k + LOOK) % RS
            pltpu.make_async_copy(h_sp.at[ring.at[k, 0]], rows.at[b],
                                  sems_g[b]).wait()
            pltpu.async_copy(rows.at[b], acc.at[ring.at[k, 1]],
                             sems_s[b], add=True)
            if not first:
                pltpu.make_async_copy(rows.at[bn],
                                      acc.at[ring.at[(k - 1) % RS, 1]],
                                      sems_s[bn]).wait()
            pltpu.async_copy(e_hbm.at[wid, j0 + k + RS - 1],
                             ring.at[(k - 1) % RS],
                             sems_e[(k - 1) % RS])
            pltpu.make_async_copy(e_hbm.at[wid, j0 + k + LOOK], ring.at[rn],
                                  sems_e[rn]).wait()
            pltpu.async_copy(h_sp.at[ring.at[rn, 0]], rows.at[bn], sems_g[bn])

        for p in range(npass):
            pltpu.sync_copy(h_hbm.at[p, sl], h_sp.at[sl])
            pltpu.sync_copy(zeros_hbm.at[sl], acc.at[sl])
            for k in range(RS - 1):
                pltpu.async_copy(e_hbm.at[wid, k], ring.at[k], sems_e[k])
            plsc.subcore_barrier()
            for b in range(LOOK):
                pltpu.make_async_copy(e_hbm.at[wid, b], ring.at[b],
                                      sems_e[b]).wait()
                pltpu.async_copy(h_sp.at[ring.at[b, 0]], rows.at[b], sems_g[b])

            for k in range(RS):
                phase(0, k, first=(k == 0))

            def body(u, carry):
                j0 = u * RS
                for k in range(RS):
                    phase(j0, k, first=False)
                return carry

            lax.fori_loop(1, C_REAL // RS, body, 0)

            for i in range(LOOK):
                b = (C_REAL + i) % NBUF
                pltpu.make_async_copy(h_sp.at[ring.at[b, 0]], rows.at[b],
                                      sems_g[b]).wait()
            bl = (C_REAL - 1) % NBUF
            pltpu.make_async_copy(rows.at[bl], acc.at[ring.at[0, 1]],
                                  sems_s[bl]).wait()
            for i in range(LOOK, RS - 1):
                k = (C_REAL + i) % RS
                pltpu.make_async_copy(e_hbm.at[wid, k], ring.at[k],
                                      sems_e[k]).wait()
            plsc.subcore_barrier()
            pltpu.sync_copy(acc.at[sl], out_hbm.at[c, p, sl])

    params = None
    if d % 128 != 0:
        params = pltpu.CompilerParams(use_tc_tiling_on_sc=False)
    return pl.kernel(
        _agg_body,
        out_type=jax.ShapeDtypeStruct((NC, npass, NPAD, d), jnp.float32),
        mesh=_mesh,
        compiler_params=params,
        scratch_types=[
            pltpu.VMEM((RS, 2, CHUNK), jnp.int32),
            pltpu.VMEM((NBUF, CHUNK, d), jnp.float32),
            pltpu.VMEM_SHARED((NPAD, d), jnp.float32),
            pltpu.VMEM_SHARED((NPAD, d), jnp.float32),
        ] + [pltpu.SemaphoreType.DMA] * (RS + 2 * NBUF),
    )


_agg64x2 = _make_agg(64, 2)
_agg40 = _make_agg(40, 1)


def _tc1_body(x_ref, w_ref, deg_ref, o_ref):
    d = jnp.sum(deg_ref[...], axis=(0, 1))
    dsum = jnp.sum(d, axis=1, keepdims=True)
    cs = lax.rsqrt(jnp.maximum(dsum, 1.0))
    h = jnp.dot(x_ref[...], w_ref[...], preferred_element_type=jnp.float32)
    h = h * cs
    o_ref[...] = jnp.stack([h[:, :64], h[:, 64:]], axis=0)


def _tc1(feats_p, w1, degs):
    return pl.pallas_call(
        _tc1_body,
        grid=(NPAD // BLK,),
        in_specs=[
            pl.BlockSpec((BLK, 128), lambda i: (i, 0)),
            pl.BlockSpec((128, 128), lambda i: (0, 0)),
            pl.BlockSpec((NC, 1, BLK, 8), lambda i: (0, 0, i, 0)),
        ],
        out_specs=pl.BlockSpec((2, BLK, 64), lambda i: (0, i, 0)),
        out_shape=jax.ShapeDtypeStruct((2, NPAD, 64), jnp.float32),
    )(feats_p, w1, degs)


def _tc2_body(a_ref, deg_ref, b1_ref, w2_ref, o_ref):
    d = deg_ref[...]
    do = jnp.sum(jnp.sum(d[:, 0], axis=0), axis=1, keepdims=True)
    di = jnp.sum(jnp.sum(d[:, 1], axis=0), axis=1, keepdims=True)
    cs = lax.rsqrt(jnp.maximum(do, 1.0))
    ci = lax.rsqrt(jnp.maximum(di, 1.0))
    a = a_ref[...]
    agg = jnp.concatenate([a[0, 0] + a[1, 0], a[0, 1] + a[1, 1]], axis=-1)
    t = jnp.maximum(agg * ci + b1_ref[...], 0.0)
    h2 = jnp.dot(t, w2_ref[...], preferred_element_type=jnp.float32)
    o_ref[...] = h2 * cs


def _tc2(a1, degs, b1r, w2):
    return pl.pallas_call(
        _tc2_body,
        grid=(NPAD // BLK,),
        in_specs=[
            pl.BlockSpec((NC, 2, BLK, 64), lambda i: (0, 0, i, 0)),
            pl.BlockSpec((NC, 2, BLK, 8), lambda i: (0, 0, i, 0)),
            pl.BlockSpec((1, 128), lambda i: (0, 0)),
            pl.BlockSpec((128, 40), lambda i: (0, 0)),
        ],
        out_specs=pl.BlockSpec((BLK, 40), lambda i: (i, 0)),
        out_shape=jax.ShapeDtypeStruct((NPAD, 40), jnp.float32),
    )(a1, degs, b1r, w2)


def _tc3_body(a_ref, deg_ref, b2_ref, o_ref):
    d = jnp.sum(deg_ref[...], axis=(0, 1))
    dsum = jnp.sum(d, axis=1, keepdims=True)
    ci = lax.rsqrt(jnp.maximum(dsum, 1.0))
    a = a_ref[...]
    o_ref[...] = (a[0, 0] + a[1, 0]) * ci + b2_ref[...]


def _tc3(a2, degs, b2r):
    return pl.pallas_call(
        _tc3_body,
        grid=(NPAD // BLK,),
        in_specs=[
            pl.BlockSpec((NC, 1, BLK, 40), lambda i: (0, 0, i, 0)),
            pl.BlockSpec((NC, 1, BLK, 8), lambda i: (0, 1, i, 0)),
            pl.BlockSpec((1, 40), lambda i: (0, 0)),
        ],
        out_specs=pl.BlockSpec((BLK, 40), lambda i: (i, 0)),
        out_shape=jax.ShapeDtypeStruct((NPAD, 40), jnp.float32),
    )(a2, degs, b2r)


def _pack_edges(e32):
    pad_n = C_REAL * NW * CHUNK - N_EDGES
    pad = jnp.full((2, pad_n), DUMMY, jnp.int32)
    main = jnp.concatenate([e32, pad], axis=1).reshape(2, NW, C_REAL, CHUNK)
    main = jnp.transpose(main, (1, 2, 0, 3))
    filler = jnp.full((NW, C_ARR - C_REAL, 2, CHUNK), DUMMY, jnp.int32)
    return jnp.concatenate([main, filler], axis=1)


def kernel(feats, edge_index, W1, b1, W2, b2):
    e3 = _pack_edges(edge_index.astype(jnp.int32))

    feats_p = jnp.pad(feats, ((0, NPAD - N_NODES), (0, 0)))
    b1r = b1.reshape(1, 128)
    b2r = b2.reshape(1, 40)

    ones8 = jnp.tile(jnp.array([1, 0, 0, 0, 0, 0, 0, 0], jnp.float32), (CHUNK, 1))
    zeros8 = jnp.zeros((NPAD, 8), jnp.float32)
    zeros64 = jnp.zeros((NPAD, 64), jnp.float32)
    zeros40 = jnp.zeros((NPAD, 40), jnp.float32)

    degs = _deg_call(e3, ones8, zeros8)
    h1 = _tc1(feats_p, W1, degs)
    a1 = _agg64x2(h1, e3, zeros64)
    h2 = _tc2(a1, degs, b1r, W2)
    a2 = _agg40(h2.reshape(1, NPAD, 40), e3, zeros40)
    out = _tc3(a2, degs, b2r)
    return out[:N_NODES]


# speedup vs baseline: 1.0665x; 1.0503x over previous
import jax
import jax.numpy as jnp
from jax import lax
from jax.experimental import pallas as pl
from jax.experimental.pallas import tpu as pltpu
from jax.experimental.pallas import tpu_sc as plsc

N_NODES = 10000
N_EDGES = 320000
NPAD = 10240
DUMMY = N_NODES
NC, NS = 2, 16
NW = NC * NS
CHUNK = 128
C_REAL = 80
NBUF = 4
RS = 8
LOOK = 3
C_ARR = C_REAL + RS
RPT = NPAD // NS
BLK = 1024

_mesh = plsc.VectorSubcoreMesh(core_axis_name="c", subcore_axis_name="s")


def _deg_body(e_hbm, ones_hbm, zeros_hbm, out_hbm,
              e_v, ones_v, acc_o, acc_i, sem_s):
    c = lax.axis_index("c")
    s = lax.axis_index("s")
    wid = s * NC + c
    pltpu.sync_copy(e_hbm.at[wid, pl.ds(0, C_REAL)], e_v)
    pltpu.sync_copy(ones_hbm, ones_v)
    sl = pl.ds(s * RPT, RPT)
    pltpu.sync_copy(zeros_hbm.at[sl], acc_o.at[sl])
    pltpu.sync_copy(zeros_hbm.at[sl], acc_i.at[sl])
    plsc.subcore_barrier()

    def fire(j, carry):
        pltpu.async_copy(ones_v, acc_o.at[e_v.at[j, 0]], sem_s, add=True)
        pltpu.async_copy(ones_v, acc_i.at[e_v.at[j, 1]], sem_s, add=True)
        return carry

    lax.fori_loop(0, C_REAL, fire, 0)

    def drain(j, carry):
        pltpu.make_async_copy(ones_v, acc_o.at[e_v.at[j, 0]], sem_s).wait()
        pltpu.make_async_copy(ones_v, acc_i.at[e_v.at[j, 1]], sem_s).wait()
        return carry

    lax.fori_loop(0, C_REAL, drain, 0)
    plsc.subcore_barrier()
    pltpu.sync_copy(acc_o.at[sl], out_hbm.at[c, 0, sl])
    pltpu.sync_copy(acc_i.at[sl], out_hbm.at[c, 1, sl])


_deg_call = pl.kernel(
    _deg_body,
    out_type=jax.ShapeDtypeStruct((NC, 2, NPAD, 8), jnp.float32),
    mesh=_mesh,
    scratch_types=[
        pltpu.VMEM((C_REAL, 2, CHUNK), jnp.int32),
        pltpu.VMEM((CHUNK, 8), jnp.float32),
        pltpu.VMEM_SHARED((NPAD, 8), jnp.float32),
        pltpu.VMEM_SHARED((NPAD, 8), jnp.float32),
        pltpu.SemaphoreType.DMA,
    ],
)


def _make_agg(d, npass):
    def _agg_body(h_hbm, e_hbm, zeros_hbm, out_hbm,
                  ring, rows, h_sp, acc,
                  se0, se1, se2, se3, se4, se5, se6, se7,
                  sg0, sg1, sg2, sg3, ss0, ss1, ss2, ss3):
        c = lax.axis_index("c")
        s = lax.axis_index("s")
        wid = s * NC + c
        sems_e = (se0, se1, se2, se3, se4, se5, se6, se7)
        sems_g = (sg0, sg1, sg2, sg3)
        sems_s = (ss0, ss1, ss2, ss3)
        sl = pl.ds(s * RPT, RPT)

        def phase(j0, k, first):
            b = k % NBUF
            bn = (k + LOOK) % NBUF
            rn = (k + LOOK) % RS
            pltpu.make_async_copy(h_sp.at[ring.at[k, 0]], rows.at[b],
                                  sems_g[b]).wait()
            pltpu.async_copy(rows.at[b], acc.at[ring.at[k, 1]],
                             sems_s[b], add=True)
            if not first:
                pltpu.make_async_copy(rows.at[bn],
                                      acc.at[ring.at[(k - 1) % RS, 1]],
                                      sems_s[bn]).wait()
            pltpu.async_copy(e_hbm.at[wid, j0 + k + RS - 1],
                             ring.at[(k - 1) % RS],
                             sems_e[(k - 1) % RS])
            pltpu.make_async_copy(e_hbm.at[wid, j0 + k + LOOK], ring.at[rn],
                                  sems_e[rn]).wait()
            pltpu.async_copy(h_sp.at[ring.at[rn, 0]], rows.at[bn], sems_g[bn])

        for p in range(npass):
            pltpu.sync_copy(h_hbm.at[p, sl], h_sp.at[sl])
            pltpu.sync_copy(zeros_hbm.at[sl], acc.at[sl])
            for k in range(RS - 1):
                pltpu.async_copy(e_hbm.at[wid, k], ring.at[k], sems_e[k])
            plsc.subcore_barrier()
            for b in range(LOOK):
                pltpu.make_async_copy(e_hbm.at[wid, b], ring.at[b],
                                      sems_e[b]).wait()
                pltpu.async_copy(h_sp.at[ring.at[b, 0]], rows.at[b], sems_g[b])

            for k in range(RS):
                phase(0, k, first=(k == 0))

            def body(u, carry):
                j0 = u * RS
                for k in range(RS):
                    phase(j0, k, first=False)
                return carry

            lax.fori_loop(1, C_REAL // RS, body, 0)

            for i in range(LOOK):
                b = (C_REAL + i) % NBUF
                pltpu.make_async_copy(h_sp.at[ring.at[b, 0]], rows.at[b],
                                      sems_g[b]).wait()
            bl = (C_REAL - 1) % NBUF
            pltpu.make_async_copy(rows.at[bl], acc.at[ring.at[0, 1]],
                                  sems_s[bl]).wait()
            for i in range(LOOK, RS - 1):
                k = (C_REAL + i) % RS
                pltpu.make_async_copy(e_hbm.at[wid, k], ring.at[k],
                                      sems_e[k]).wait()
            plsc.subcore_barrier()
            pltpu.sync_copy(acc.at[sl], out_hbm.at[c, p, sl])

    params = None
    if d % 128 != 0:
        params = pltpu.CompilerParams(use_tc_tiling_on_sc=False)
    return pl.kernel(
        _agg_body,
        out_type=jax.ShapeDtypeStruct((NC, npass, NPAD, d), jnp.float32),
        mesh=_mesh,
        compiler_params=params,
        scratch_types=[
            pltpu.VMEM((RS, 2, CHUNK), jnp.int32),
            pltpu.VMEM((NBUF, CHUNK, d), jnp.float32),
            pltpu.VMEM_SHARED((NPAD, d), jnp.float32),
            pltpu.VMEM_SHARED((NPAD, d), jnp.float32),
        ] + [pltpu.SemaphoreType.DMA] * (RS + 2 * NBUF),
    )


_agg64x2 = _make_agg(64, 2)
_agg40 = _make_agg(40, 1)


def _tc1_body(x_ref, w_ref, deg_ref, o_ref):
    d = jnp.sum(deg_ref[...], axis=(0, 1))
    dsum = jnp.sum(d, axis=1, keepdims=True)
    cs = lax.rsqrt(jnp.maximum(dsum, 1.0))
    h = jnp.dot(x_ref[...], w_ref[...], preferred_element_type=jnp.float32)
    h = h * cs
    o_ref[...] = jnp.stack([h[:, :64], h[:, 64:]], axis=0)


def _tc1(feats_p, w1, degs):
    return pl.pallas_call(
        _tc1_body,
        grid=(NPAD // BLK,),
        in_specs=[
            pl.BlockSpec((BLK, 128), lambda i: (i, 0)),
            pl.BlockSpec((128, 128), lambda i: (0, 0)),
            pl.BlockSpec((NC, 1, BLK, 8), lambda i: (0, 0, i, 0)),
        ],
        out_specs=pl.BlockSpec((2, BLK, 64), lambda i: (0, i, 0)),
        out_shape=jax.ShapeDtypeStruct((2, NPAD, 64), jnp.float32),
    )(feats_p, w1, degs)


def _tc2_body(a_ref, deg_ref, b1_ref, w2_ref, o_ref):
    d = deg_ref[...]
    do = jnp.sum(jnp.sum(d[:, 0], axis=0), axis=1, keepdims=True)
    di = jnp.sum(jnp.sum(d[:, 1], axis=0), axis=1, keepdims=True)
    cs = lax.rsqrt(jnp.maximum(do, 1.0))
    ci = lax.rsqrt(jnp.maximum(di, 1.0))
    a = a_ref[...]
    agg = jnp.concatenate([a[0, 0] + a[1, 0], a[0, 1] + a[1, 1]], axis=-1)
    t = jnp.maximum(agg * ci + b1_ref[...], 0.0)
    h2 = jnp.dot(t, w2_ref[...], preferred_element_type=jnp.float32)
    o_ref[...] = h2 * cs


def _tc2(a1, degs, b1r, w2):
    return pl.pallas_call(
        _tc2_body,
        grid=(NPAD // BLK,),
        in_specs=[
            pl.BlockSpec((NC, 2, BLK, 64), lambda i: (0, 0, i, 0)),
            pl.BlockSpec((NC, 2, BLK, 8), lambda i: (0, 0, i, 0)),
            pl.BlockSpec((1, 128), lambda i: (0, 0)),
            pl.BlockSpec((128, 40), lambda i: (0, 0)),
        ],
        out_specs=pl.BlockSpec((BLK, 40), lambda i: (i, 0)),
        out_shape=jax.ShapeDtypeStruct((NPAD, 40), jnp.float32),
    )(a1, degs, b1r, w2)


def _tc3_body(a_ref, deg_ref, b2_ref, o_ref):
    d = jnp.sum(deg_ref[...], axis=(0, 1))
    dsum = jnp.sum(d, axis=1, keepdims=True)
    ci = lax.rsqrt(jnp.maximum(dsum, 1.0))
    a = a_ref[...]
    o_ref[...] = (a[0, 0] + a[1, 0]) * ci + b2_ref[...]


def _tc3(a2, degs, b2r):
    return pl.pallas_call(
        _tc3_body,
        grid=(NPAD // BLK,),
        in_specs=[
            pl.BlockSpec((NC, 1, BLK, 40), lambda i: (0, 0, i, 0)),
            pl.BlockSpec((NC, 1, BLK, 8), lambda i: (0, 1, i, 0)),
            pl.BlockSpec((1, 40), lambda i: (0, 0)),
        ],
        out_specs=pl.BlockSpec((BLK, 40), lambda i: (i, 0)),
        out_shape=jax.ShapeDtypeStruct((NPAD, 40), jnp.float32),
    )(a2, degs, b2r)


def _pack_edges(e32):
    pad_n = C_REAL * NW * CHUNK - N_EDGES
    pad = jnp.full((2, pad_n), DUMMY, jnp.int32)
    main = jnp.concatenate([e32, pad], axis=1).reshape(2, NW, C_REAL, CHUNK)
    main = jnp.transpose(main, (1, 2, 0, 3))
    filler = jnp.full((NW, C_ARR - C_REAL, 2, CHUNK), DUMMY, jnp.int32)
    return jnp.concatenate([main, filler], axis=1)


def kernel(feats, edge_index, W1, b1, W2, b2):
    e3 = _pack_edges(edge_index.astype(jnp.int32))

    feats_p = jnp.pad(feats, ((0, NPAD - N_NODES), (0, 0)))
    b1r = b1.reshape(1, 128)
    b2r = b2.reshape(1, 40)

    ones8 = jnp.tile(jnp.array([1, 0, 0, 0, 0, 0, 0, 0], jnp.float32), (CHUNK, 1))
    zeros8 = jnp.zeros((NPAD, 8), jnp.float32)
    zeros64 = jnp.zeros((NPAD, 64), jnp.float32)
    zeros40 = jnp.zeros((NPAD, 40), jnp.float32)

    degs = _deg_call(e3, ones8, zeros8)
    h1 = _tc1(feats_p, W1, degs)
    a1 = _agg64x2(h1, e3, zeros64)
    h2 = _tc2(a1, degs, b1r, W2)
    a2 = _agg40(h2.reshape(1, NPAD, 40), e3, zeros40)
    out = _tc3(a2, degs, b2r)
    return out[:N_NODES]
